# baseline (speedup 1.0000x reference)
import functools

import jax
import jax.numpy as jnp
from jax import lax
from jax.experimental import pallas as pl
from jax.experimental.pallas import tpu as pltpu
from jax.experimental.pallas import tpu_sc as plsc

_B = 16384
_K = 3
_H = 2048
_N = _B * _K

_info = plsc.get_sparse_core_info()
_NC, _NS, _L = _info.num_cores, _info.num_subcores, _info.num_lanes
_NW = _NC * _NS
_BPW = _N // _NW
_G = 16
_NB = _BPW // _G
_S = 4

_mesh = plsc.VectorSubcoreMesh(core_axis_name="c", subcore_axis_name="s")


@functools.partial(
    pl.kernel,
    mesh=_mesh,
    out_type=jax.ShapeDtypeStruct((_N, _H), jnp.float32),
    scratch_types=[
        pltpu.VMEM((_BPW,), jnp.int32),
        pltpu.VMEM((3, _H), jnp.float32),
        pltpu.VMEM((_G, _H), jnp.float32),
    ] + [pltpu.SemaphoreType.DMA] * _S,
)
def _lookup(idx_hbm, table_hbm, out_hbm, idx_v, table_v, dummy_v, *sems):
    wid = lax.axis_index("s") * _NC + lax.axis_index("c")
    base = wid * _BPW
    pltpu.sync_copy(table_hbm, table_v)
    pltpu.sync_copy(idx_hbm.at[pl.ds(base, _BPW)], idx_v)

    def _issue(b, sem):
        v = idx_v[pl.ds(b * _G, _G)]
        v = jnp.where(v < 0, 2, v)
        for j in range(_G):
            r = v[j]
            pltpu.async_copy(
                table_v.at[pl.ds(r, 1)],
                out_hbm.at[pl.ds(base + b * _G + j, 1)],
                sem,
            )

    def _drain(sem):
        pltpu.make_async_copy(out_hbm.at[pl.ds(0, _G)], dummy_v, sem).wait()

    for k in range(_S):
        _issue(k, sems[k])

    def _body(q, carry):
        b = _S * q
        for k in range(_S):
            _drain(sems[k])
            _issue(b + k + _S, sems[k])
        return carry

    lax.fori_loop(0, _NB // _S - 1, _body, 0)
    for k in range(_S):
        _drain(sems[k])


def kernel(arm_labels, embedding_weight):
    idx = arm_labels.reshape(_N).astype(jnp.int32)
    out = _lookup(idx, embedding_weight)
    return out.reshape(_B, _K, _H)


# speedup vs baseline: 2.2943x; 2.2943x over previous
import functools

import jax
import jax.numpy as jnp
from jax import lax
from jax.experimental import pallas as pl
from jax.experimental.pallas import tpu as pltpu
from jax.experimental.pallas import tpu_sc as plsc

_B = 16384
_K = 3
_H = 2048
_N = _B * _K

_info = plsc.get_sparse_core_info()
_NC, _NS, _L = _info.num_cores, _info.num_subcores, _info.num_lanes
_NW = _NC * _NS
_BPW = _N // _NW
_G = 16
_NB = _BPW // _G
_S = 4
_DRAIN = _G * _H

_mesh = plsc.VectorSubcoreMesh(core_axis_name="c", subcore_axis_name="s")


@functools.partial(
    pl.kernel,
    mesh=_mesh,
    out_type=jax.ShapeDtypeStruct((_B, _K, _H), jnp.float32),
    scratch_types=[
        pltpu.VMEM((_BPW,), jnp.int32),
        pltpu.VMEM((_K, _H), jnp.float32),
        pltpu.VMEM((_DRAIN,), jnp.int32),
    ] + [pltpu.SemaphoreType.DMA] * _S,
)
def _lookup(idx_hbm, table_hbm, out_hbm, idx_v, table_v, dummy_v, *sems):
    wid = lax.axis_index("s") * _NC + lax.axis_index("c")
    base = wid * _BPW
    pltpu.sync_copy(table_hbm, table_v)
    pltpu.sync_copy(idx_hbm.at[pl.ds(base, _BPW)], idx_v)

    def _issue(b, sem):
        v = idx_v[pl.ds(b * _G, _G)]
        v = jnp.where(v < 0, 2, v)
        for j in range(_G):
            r = v[j]
            row = base + b * _G + j
            pltpu.async_copy(
                table_v.at[r],
                out_hbm.at[row // _K, row % _K],
                sem,
            )

    def _drain(sem):
        pltpu.make_async_copy(
            idx_hbm.at[pl.ds(0, _DRAIN)], dummy_v, sem
        ).wait()

    for k in range(_S):
        _issue(k, sems[k])

    def _body(q, carry):
        b = _S * q
        for k in range(_S):
            _drain(sems[k])
            _issue(b + k + _S, sems[k])
        return carry

    lax.fori_loop(0, _NB // _S - 1, _body, 0)
    for k in range(_S):
        _drain(sems[k])


def kernel(arm_labels, embedding_weight):
    idx = arm_labels.reshape(_N).astype(jnp.int32)
    return _lookup(idx, embedding_weight)


# speedup vs baseline: 6.1296x; 2.6716x over previous
import functools

import jax
import jax.numpy as jnp
from jax import lax
from jax.experimental import pallas as pl
from jax.experimental.pallas import tpu as pltpu
from jax.experimental.pallas import tpu_sc as plsc

_B = 16384
_K = 3
_H = 2048
_N = _B * _K

_info = plsc.get_sparse_core_info()
_NC, _NS, _L = _info.num_cores, _info.num_subcores, _info.num_lanes
_NW = _NC * _NS
_BPW = _N // _NW
_EPW = _B // _NW
_G = 16
_NB = _BPW // _G
_S = 4
_DRAIN = _G * _H

_mesh = plsc.VectorSubcoreMesh(core_axis_name="c", subcore_axis_name="s")


@functools.partial(
    pl.kernel,
    mesh=_mesh,
    out_type=jax.ShapeDtypeStruct((_K, _B, _H), jnp.float32),
    scratch_types=[
        pltpu.VMEM((_BPW,), jnp.int32),
        pltpu.VMEM((_K, _H), jnp.float32),
        pltpu.VMEM((_DRAIN,), jnp.int32),
    ] + [pltpu.SemaphoreType.DMA] * _S,
)
def _lookup(idx_hbm, table_hbm, out_hbm, idx_v, table_v, dummy_v, *sems):
    wid = lax.axis_index("s") * _NC + lax.axis_index("c")
    base = wid * _BPW
    ebase = wid * _EPW
    pltpu.sync_copy(table_hbm, table_v)
    pltpu.sync_copy(idx_hbm.at[pl.ds(base, _BPW)], idx_v)

    def _issue(b, sem):
        v = idx_v[pl.ds(b * _G, _G)]
        v = jnp.where(v < 0, 2, v)
        for j in range(_G):
            r = v[j]
            row = b * _G + j
            pltpu.async_copy(
                table_v.at[r],
                out_hbm.at[row % _K, ebase + row // _K],
                sem,
            )

    def _drain(sem):
        pltpu.make_async_copy(
            idx_hbm.at[pl.ds(0, _DRAIN)], dummy_v, sem
        ).wait()

    for k in range(_S):
        _issue(k, sems[k])

    def _body(q, carry):
        b = _S * q
        for k in range(_S):
            _drain(sems[k])
            _issue(b + k + _S, sems[k])
        return carry

    lax.fori_loop(0, _NB // _S - 1, _body, 0)
    for k in range(_S):
        _drain(sems[k])


def kernel(arm_labels, embedding_weight):
    idx = arm_labels.reshape(_N).astype(jnp.int32)
    out = _lookup(idx, embedding_weight)
    return jnp.transpose(out, (1, 0, 2))
